# baseline (speedup 1.0000x reference)
import functools

import jax
import jax.numpy as jnp
from jax import lax
from jax.experimental import pallas as pl
from jax.experimental.pallas import tpu as pltpu
from jax.experimental.pallas import tpu_sc as plsc


def _make_gather(V, D, B, C, U=16):
    NC, NS = 2, 16
    NW = NC * NS
    b_per_w = B // NW
    assert b_per_w % C == 0 and b_per_w * NW == B
    n = b_per_w // C
    assert n % 2 == 0 and n >= 6 and C % U == 0 and U % 2 == 0
    C2 = C // 2
    b2_per_w = b_per_w // 2
    B2 = B // 2
    D2 = 2 * D
    mesh = plsc.VectorSubcoreMesh(core_axis_name="c", subcore_axis_name="s")

    @functools.partial(
        pl.kernel,
        mesh=mesh,
        compiler_params=pltpu.CompilerParams(use_tc_tiling_on_sc=False),
        out_type=jax.ShapeDtypeStruct((B2, D2), jnp.float32),
        scratch_types=[
            pltpu.VMEM((V, D), jnp.float32),
            pltpu.VMEM((C,), jnp.int32),
            pltpu.VMEM((C,), jnp.int32),
            pltpu.VMEM((C2, D2), jnp.float32),
            pltpu.VMEM((C2, D2), jnp.float32),
            pltpu.SemaphoreType.DMA,
            pltpu.SemaphoreType.DMA,
            pltpu.SemaphoreType.DMA,
            pltpu.SemaphoreType.DMA,
        ],
    )
    def k(idx_hbm, table_hbm, out_hbm, table_v, i0, i1, r0, r1, l0, l1, w0, w1):
        ibuf = (i0, i1)
        rbuf = (r0, r1)
        lsem = (l0, l1)
        wsem = (w0, w1)
        wid = lax.axis_index("s") * NC + lax.axis_index("c")
        base0 = wid * b_per_w
        base2 = wid * b2_per_w

        def startL(i, b):
            pltpu.async_copy(idx_hbm.at[pl.ds(base0 + i * C, C)], ibuf[b], lsem[b])

        def waitL(b):
            pltpu.make_async_copy(idx_hbm.at[pl.ds(base0, C)], ibuf[b], lsem[b]).wait()

        def startW(i, b):
            pltpu.async_copy(rbuf[b], out_hbm.at[pl.ds(base2 + i * C2, C2)], wsem[b])

        def waitW(b):
            pltpu.make_async_copy(rbuf[b], out_hbm.at[pl.ds(base2, C2)], wsem[b]).wait()

        def compute(b):
            src = ibuf[b]
            dst = rbuf[b]

            @plsc.parallel_loop(0, C, step=U)
            def _(j0):
                sv = src[pl.ds(j0, U)]
                for u in range(U):
                    s = sv[u]
                    col = D * (u % 2)
                    for k2 in range(D // 16):
                        dst[j0 // 2 + u // 2, pl.ds(col + 16 * k2, 16)] = (
                            table_v[s, pl.ds(16 * k2, 16)]
                        )

        pltpu.sync_copy(table_hbm, table_v)
        startL(0, 0)
        startL(1, 1)
        for i in (0, 1):
            waitL(i)
            compute(i)
            startW(i, i)
            startL(i + 2, i)

        @pl.loop(2, n - 2, step=2)
        def _(t):
            for d in range(2):
                i = t + d
                b = d
                waitL(b)
                waitW(b)
                compute(b)
                startW(i, b)
                startL(i + 2, b)

        for i in (n - 2, n - 1):
            b = i % 2
            waitL(b)
            waitW(b)
            compute(b)
            startW(i, b)
        waitW(0)
        waitW(1)

    return k


def kernel(tc, embedding):
    B0, H = tc.shape
    V, D = embedding.shape
    B = B0 * H
    flat = tc.reshape(B).astype(jnp.int32)
    out = _make_gather(V, D, B, 512)(flat, embedding.astype(jnp.float32))
    return out.reshape(B0, H, D)


# speedup vs baseline: 1.0013x; 1.0013x over previous
import functools

import jax
import jax.numpy as jnp
from jax import lax
from jax.experimental import pallas as pl
from jax.experimental.pallas import tpu as pltpu
from jax.experimental.pallas import tpu_sc as plsc


def _make_gather(VP, DP, D, B, C, U=16):
    NC, NS = 2, 16
    NW = NC * NS
    b_per_w = B // NW
    assert b_per_w % C == 0 and b_per_w * NW == B
    n = b_per_w // C
    assert n % 2 == 0 and n >= 6 and C % U == 0 and U % 2 == 0
    C2 = C // 2
    b2_per_w = b_per_w // 2
    B2 = B // 2
    D2 = 2 * D
    mesh = plsc.VectorSubcoreMesh(core_axis_name="c", subcore_axis_name="s")

    @functools.partial(
        pl.kernel,
        mesh=mesh,
        out_type=jax.ShapeDtypeStruct((B2, D2), jnp.float32),
        scratch_types=[
            pltpu.VMEM((VP, DP), jnp.float32),
            pltpu.VMEM((C,), jnp.int32),
            pltpu.VMEM((C,), jnp.int32),
            pltpu.VMEM((C2, D2), jnp.float32),
            pltpu.VMEM((C2, D2), jnp.float32),
            pltpu.SemaphoreType.DMA,
            pltpu.SemaphoreType.DMA,
            pltpu.SemaphoreType.DMA,
            pltpu.SemaphoreType.DMA,
        ],
    )
    def k(idx_hbm, table_hbm, out_hbm, table_v, i0, i1, r0, r1, l0, l1, w0, w1):
        ibuf = (i0, i1)
        rbuf = (r0, r1)
        lsem = (l0, l1)
        wsem = (w0, w1)
        wid = lax.axis_index("s") * NC + lax.axis_index("c")
        base0 = wid * b_per_w
        base2 = wid * b2_per_w

        def startL(i, b):
            pltpu.async_copy(idx_hbm.at[pl.ds(base0 + i * C, C)], ibuf[b], lsem[b])

        def waitL(b):
            pltpu.make_async_copy(idx_hbm.at[pl.ds(base0, C)], ibuf[b], lsem[b]).wait()

        def startW(i, b):
            pltpu.async_copy(rbuf[b], out_hbm.at[pl.ds(base2 + i * C2, C2)], wsem[b])

        def waitW(b):
            pltpu.make_async_copy(rbuf[b], out_hbm.at[pl.ds(base2, C2)], wsem[b]).wait()

        def compute(b):
            src = ibuf[b]
            dst = rbuf[b]

            @plsc.parallel_loop(0, C, step=U)
            def _(j0):
                sv = src[pl.ds(j0, U)]
                for u in range(U):
                    s = sv[u]
                    col = D * (u % 2)
                    for k2 in range(D // 16):
                        dst[j0 // 2 + u // 2, pl.ds(col + 16 * k2, 16)] = (
                            table_v[s, pl.ds(16 * k2, 16)]
                        )

        pltpu.sync_copy(table_hbm, table_v)
        startL(0, 0)
        startL(1, 1)
        for i in (0, 1):
            waitL(i)
            compute(i)
            startW(i, i)
            startL(i + 2, i)

        @pl.loop(2, n - 2, step=2)
        def _(t):
            for d in range(2):
                i = t + d
                b = d
                waitL(b)
                waitW(b)
                compute(b)
                startW(i, b)
                startL(i + 2, b)

        for i in (n - 2, n - 1):
            b = i % 2
            waitL(b)
            waitW(b)
            compute(b)
            startW(i, b)
        waitW(0)
        waitW(1)

    return k


def kernel(tc, embedding):
    B0, H = tc.shape
    V, D = embedding.shape
    B = B0 * H
    VP = V + (-V) % 8
    DP = 128
    flat = tc.reshape(B).astype(jnp.int32)
    table = jnp.pad(embedding.astype(jnp.float32), ((0, VP - V), (0, DP - D)))
    out = _make_gather(VP, DP, D, B, 512)(flat, table)
    return out.reshape(B0, H, D)
